# baseline (speedup 1.0000x reference)
import jax
import jax.numpy as jnp
from jax.experimental import pallas as pl

_N = 100000
_B = 64
_K = 16
_D = 128
_BM = 4000


def _probs(x, cent, c2):
    dot = jax.lax.dot_general(
        x, cent, (((1,), (1,)), ((), ())), preferred_element_type=jnp.float32
    )
    r2 = jnp.sum(x * x, axis=1, keepdims=True)
    d2 = r2 + c2[None, :] - 2.0 * dot
    s = -jnp.sqrt(jnp.maximum(d2, 1e-12))
    m = jnp.max(s, axis=1, keepdims=True)
    e = jnp.exp(s - m)
    return e / jnp.sum(e, axis=1, keepdims=True)


def _body(res_ref, feats_ref, dc_ref, init_ref, out_big_ref, out_small_ref):
    cent = dc_ref[:] + init_ref[:]
    c2 = jnp.sum(cent * cent, axis=1)
    out_big_ref[:] = _probs(res_ref[:], cent, c2)

    @pl.when(pl.program_id(0) == 0)
    def _():
        out_small_ref[:] = _probs(feats_ref[:], cent, c2)


def kernel(feats, reservoir_feats, delta_centroids, init_style):
    grid = _N // _BM
    out_big, out_small = pl.pallas_call(
        _body,
        grid=(grid,),
        in_specs=[
            pl.BlockSpec((_BM, _D), lambda i: (i, 0)),
            pl.BlockSpec((_B, _D), lambda i: (0, 0)),
            pl.BlockSpec((_K, _D), lambda i: (0, 0)),
            pl.BlockSpec((1, _D), lambda i: (0, 0)),
        ],
        out_specs=[
            pl.BlockSpec((_BM, _K), lambda i: (i, 0)),
            pl.BlockSpec((_B, _K), lambda i: (0, 0)),
        ],
        out_shape=[
            jax.ShapeDtypeStruct((_N, _K), jnp.float32),
            jax.ShapeDtypeStruct((_B, _K), jnp.float32),
        ],
    )(reservoir_feats, feats, delta_centroids, init_style)
    return jnp.concatenate([out_big, out_small], axis=0)


# speedup vs baseline: 1.1321x; 1.1321x over previous
import jax
import jax.numpy as jnp
from jax.experimental import pallas as pl

_N = 100000
_B = 64
_K = 16
_D = 128
_BM = 4000


def _probs_t(x, cent, c2):
    dot_t = jax.lax.dot_general(
        cent, x, (((1,), (1,)), ((), ())), preferred_element_type=jnp.float32
    )
    ones = jnp.ones((1, _D), dtype=jnp.float32)
    r2_t = jax.lax.dot_general(
        ones, x * x, (((1,), (1,)), ((), ())), preferred_element_type=jnp.float32
    )
    d2_t = r2_t + c2 - 2.0 * dot_t
    s_t = -jnp.sqrt(jnp.maximum(d2_t, 1e-12))
    m_t = jnp.max(s_t, axis=0, keepdims=True)
    e_t = jnp.exp(s_t - m_t)
    p_t = e_t * (1.0 / jnp.sum(e_t, axis=0, keepdims=True))
    return p_t.T


def _body(res_ref, feats_ref, dc_ref, init_ref, out_big_ref, out_small_ref):
    cent = dc_ref[:] + init_ref[:]
    c2 = jnp.sum(cent * cent, axis=1, keepdims=True)
    out_big_ref[:] = _probs_t(res_ref[:], cent, c2)

    @pl.when(pl.program_id(0) == 0)
    def _():
        out_small_ref[:] = _probs_t(feats_ref[:], cent, c2)


def kernel(feats, reservoir_feats, delta_centroids, init_style):
    grid = _N // _BM
    out_big, out_small = pl.pallas_call(
        _body,
        grid=(grid,),
        in_specs=[
            pl.BlockSpec((_BM, _D), lambda i: (i, 0)),
            pl.BlockSpec((_B, _D), lambda i: (0, 0)),
            pl.BlockSpec((_K, _D), lambda i: (0, 0)),
            pl.BlockSpec((1, _D), lambda i: (0, 0)),
        ],
        out_specs=[
            pl.BlockSpec((_BM, _K), lambda i: (i, 0)),
            pl.BlockSpec((_B, _K), lambda i: (0, 0)),
        ],
        out_shape=[
            jax.ShapeDtypeStruct((_N, _K), jnp.float32),
            jax.ShapeDtypeStruct((_B, _K), jnp.float32),
        ],
    )(reservoir_feats, feats, delta_centroids, init_style)
    return jnp.concatenate([out_big, out_small], axis=0)


# speedup vs baseline: 1.2179x; 1.0757x over previous
import jax
import jax.numpy as jnp
from jax.experimental import pallas as pl

_N = 100000
_B = 64
_K = 16
_D = 128
_BM = 4000


def _probs_t(x, cent, c2):
    dot_t = jax.lax.dot_general(
        cent, x, (((1,), (1,)), ((), ())), preferred_element_type=jnp.float32
    )
    ones = jnp.ones((1, _D), dtype=jnp.float32)
    r2_t = jax.lax.dot_general(
        ones, x * x, (((1,), (1,)), ((), ())), preferred_element_type=jnp.float32
    )
    d2_t = r2_t + c2 - 2.0 * dot_t
    s_t = -jnp.sqrt(jnp.maximum(d2_t, 1e-12))
    m_t = jnp.max(s_t, axis=0, keepdims=True)
    e_t = jnp.exp(s_t - m_t)
    p_t = e_t * (1.0 / jnp.sum(e_t, axis=0, keepdims=True))
    return p_t.T


def _body(res_ref, feats_ref, dc_ref, init_ref, out_ref):
    i = pl.program_id(0)
    cent = dc_ref[:] + init_ref[:]
    c2 = jnp.sum(cent * cent, axis=1, keepdims=True)

    @pl.when(i < _N // _BM)
    def _():
        out_ref[:] = _probs_t(res_ref[:], cent, c2)

    @pl.when(i == _N // _BM)
    def _():
        out_ref[0:_B, :] = _probs_t(feats_ref[:], cent, c2)


def kernel(feats, reservoir_feats, delta_centroids, init_style):
    grid = _N // _BM + 1
    return pl.pallas_call(
        _body,
        grid=(grid,),
        in_specs=[
            pl.BlockSpec((_BM, _D), lambda i: (jnp.minimum(i, _N // _BM - 1), 0)),
            pl.BlockSpec((_B, _D), lambda i: (0, 0)),
            pl.BlockSpec((_K, _D), lambda i: (0, 0)),
            pl.BlockSpec((1, _D), lambda i: (0, 0)),
        ],
        out_specs=pl.BlockSpec((_BM, _K), lambda i: (i, 0)),
        out_shape=jax.ShapeDtypeStruct((_N + _B, _K), jnp.float32),
    )(reservoir_feats, feats, delta_centroids, init_style)


# speedup vs baseline: 2.7400x; 2.2498x over previous
import jax
import jax.numpy as jnp
from jax.experimental import pallas as pl

_N = 100000
_B = 64
_K = 16
_D = 128
_BM = 4096
_LAST = _N // _BM
_TAIL = _N - _LAST * _BM


def _probs_t(x, cent, c2):
    dot_t = jax.lax.dot_general(
        cent, x, (((1,), (1,)), ((), ())), preferred_element_type=jnp.float32
    )
    ones = jnp.ones((1, _D), dtype=jnp.float32)
    r2_t = jax.lax.dot_general(
        ones, x * x, (((1,), (1,)), ((), ())), preferred_element_type=jnp.float32
    )
    d2_t = r2_t + c2 - 2.0 * dot_t
    s_t = -jnp.sqrt(jnp.maximum(d2_t, 1e-12))
    m_t = jnp.max(s_t, axis=0, keepdims=True)
    e_t = jnp.exp(s_t - m_t)
    return e_t * (1.0 / jnp.sum(e_t, axis=0, keepdims=True))


def _body(res_ref, feats_ref, dc_ref, init_ref, out_ref):
    i = pl.program_id(0)
    cent = dc_ref[:] + init_ref[:]
    c2 = jnp.sum(cent * cent, axis=1, keepdims=True)

    @pl.when(i < _LAST)
    def _():
        out_ref[:] = _probs_t(res_ref[:], cent, c2)

    @pl.when(i == _LAST)
    def _():
        x = jnp.concatenate(
            [
                res_ref[0:_TAIL, :],
                feats_ref[:],
                jnp.zeros((_BM - _TAIL - _B, _D), jnp.float32),
            ],
            axis=0,
        )
        out_ref[:] = _probs_t(x, cent, c2)


def kernel(feats, reservoir_feats, delta_centroids, init_style):
    grid = _LAST + 1
    out_t = pl.pallas_call(
        _body,
        grid=(grid,),
        in_specs=[
            pl.BlockSpec((_BM, _D), lambda i: (i, 0)),
            pl.BlockSpec((_B, _D), lambda i: (0, 0)),
            pl.BlockSpec((_K, _D), lambda i: (0, 0)),
            pl.BlockSpec((1, _D), lambda i: (0, 0)),
        ],
        out_specs=pl.BlockSpec((_K, _BM), lambda i: (0, i)),
        out_shape=jax.ShapeDtypeStruct((_K, _N + _B), jnp.float32),
    )(reservoir_feats, feats, delta_centroids, init_style)
    return out_t.T


# speedup vs baseline: 2.7543x; 1.0052x over previous
import jax
import jax.numpy as jnp
from jax.experimental import pallas as pl
from jax.experimental.pallas import tpu as pltpu

_N = 100000
_B = 64
_K = 16
_D = 128
_BM = 4096
_LAST = _N // _BM
_TAIL = _N - _LAST * _BM


def _probs_t(x, cent, c2):
    dot_t = jax.lax.dot_general(
        cent, x, (((1,), (1,)), ((), ())), preferred_element_type=jnp.float32
    )
    ones = jnp.ones((1, _D), dtype=jnp.float32)
    r2_t = jax.lax.dot_general(
        ones, x * x, (((1,), (1,)), ((), ())), preferred_element_type=jnp.float32
    )
    d2_t = r2_t + c2 - 2.0 * dot_t
    s_t = -jnp.sqrt(jnp.maximum(d2_t, 1e-12))
    m_t = jnp.max(s_t, axis=0, keepdims=True)
    e_t = jnp.exp(s_t - m_t)
    return e_t * (1.0 / jnp.sum(e_t, axis=0, keepdims=True))


def _body(res_ref, feats_ref, dc_ref, init_ref, out_ref):
    i = pl.program_id(0)
    cent = dc_ref[:] + init_ref[:]
    c2 = jnp.sum(cent * cent, axis=1, keepdims=True)

    @pl.when(i < _LAST)
    def _():
        out_ref[:] = _probs_t(res_ref[:], cent, c2)

    @pl.when(i == _LAST)
    def _():
        x = jnp.concatenate(
            [
                res_ref[0:_TAIL, :],
                feats_ref[:],
                jnp.zeros((_BM - _TAIL - _B, _D), jnp.float32),
            ],
            axis=0,
        )
        out_ref[:] = _probs_t(x, cent, c2)


def kernel(feats, reservoir_feats, delta_centroids, init_style):
    grid = _LAST + 1
    out_t = pl.pallas_call(
        _body,
        grid=(grid,),
        in_specs=[
            pl.BlockSpec((_BM, _D), lambda i: (i, 0)),
            pl.BlockSpec((_B, _D), lambda i: (0, 0)),
            pl.BlockSpec((_K, _D), lambda i: (0, 0)),
            pl.BlockSpec((1, _D), lambda i: (0, 0)),
        ],
        out_specs=pl.BlockSpec((_K, _BM), lambda i: (0, i)),
        out_shape=jax.ShapeDtypeStruct((_K, _N + _B), jnp.float32),
        compiler_params=pltpu.CompilerParams(
            dimension_semantics=(pltpu.PARALLEL,)
        ),
    )(reservoir_feats, feats, delta_centroids, init_style)
    return out_t.T


# speedup vs baseline: 3.5188x; 1.2776x over previous
import jax
import jax.numpy as jnp
from jax.experimental import pallas as pl
from jax.experimental.pallas import tpu as pltpu

_N = 100000
_B = 64
_K = 16
_D = 128
_BM = 8192
_LAST = _N // _BM
_TAIL = _N - _LAST * _BM


def _probs_t(x, cent, c2):
    dot_t = jax.lax.dot_general(
        cent, x, (((1,), (1,)), ((), ())), preferred_element_type=jnp.float32
    )
    ones = jnp.ones((1, _D), dtype=jnp.float32)
    r2_t = jax.lax.dot_general(
        ones, x * x, (((1,), (1,)), ((), ())), preferred_element_type=jnp.float32
    )
    d2_t = r2_t + c2 - 2.0 * dot_t
    s_t = -jnp.sqrt(jnp.maximum(d2_t, 1e-12))
    m_t = jnp.max(s_t, axis=0, keepdims=True)
    e_t = jnp.exp(s_t - m_t)
    return e_t * (1.0 / jnp.sum(e_t, axis=0, keepdims=True))


def _body(res_ref, feats_ref, dc_ref, init_ref, out_ref):
    i = pl.program_id(0)
    cent = dc_ref[:] + init_ref[:]
    c2 = jnp.sum(cent * cent, axis=1, keepdims=True)

    @pl.when(i < _LAST)
    def _():
        out_ref[:] = _probs_t(res_ref[:], cent, c2)

    @pl.when(i == _LAST)
    def _():
        x = jnp.concatenate(
            [
                res_ref[0:_TAIL, :],
                feats_ref[:],
                jnp.zeros((_BM - _TAIL - _B, _D), jnp.float32),
            ],
            axis=0,
        )
        out_ref[:] = _probs_t(x, cent, c2)


def kernel(feats, reservoir_feats, delta_centroids, init_style):
    grid = _LAST + 1
    out_t = pl.pallas_call(
        _body,
        grid=(grid,),
        in_specs=[
            pl.BlockSpec((_BM, _D), lambda i: (i, 0)),
            pl.BlockSpec((_B, _D), lambda i: (0, 0)),
            pl.BlockSpec((_K, _D), lambda i: (0, 0)),
            pl.BlockSpec((1, _D), lambda i: (0, 0)),
        ],
        out_specs=pl.BlockSpec((_K, _BM), lambda i: (0, i)),
        out_shape=jax.ShapeDtypeStruct((_K, _N + _B), jnp.float32),
        compiler_params=pltpu.CompilerParams(
            dimension_semantics=(pltpu.PARALLEL,)
        ),
    )(reservoir_feats, feats, delta_centroids, init_style)
    return out_t.T


# speedup vs baseline: 3.9650x; 1.1268x over previous
import jax
import jax.numpy as jnp
from jax.experimental import pallas as pl
from jax.experimental.pallas import tpu as pltpu

_N = 100000
_B = 64
_K = 16
_D = 128
_BM = 16384
_LAST = _N // _BM
_TAIL = _N - _LAST * _BM


def _probs_t(x, cent, c2):
    dot_t = jax.lax.dot_general(
        cent, x, (((1,), (1,)), ((), ())), preferred_element_type=jnp.float32
    )
    ones = jnp.ones((1, _D), dtype=jnp.float32)
    r2_t = jax.lax.dot_general(
        ones, x * x, (((1,), (1,)), ((), ())), preferred_element_type=jnp.float32
    )
    d2_t = r2_t + c2 - 2.0 * dot_t
    s_t = -jnp.sqrt(jnp.maximum(d2_t, 1e-12))
    m_t = jnp.max(s_t, axis=0, keepdims=True)
    e_t = jnp.exp(s_t - m_t)
    return e_t * (1.0 / jnp.sum(e_t, axis=0, keepdims=True))


def _body(res_ref, feats_ref, dc_ref, init_ref, out_ref):
    i = pl.program_id(0)
    cent = dc_ref[:] + init_ref[:]
    c2 = jnp.sum(cent * cent, axis=1, keepdims=True)

    @pl.when(i < _LAST)
    def _():
        out_ref[:] = _probs_t(res_ref[:], cent, c2)

    @pl.when(i == _LAST)
    def _():
        x = jnp.concatenate(
            [
                res_ref[0:_TAIL, :],
                feats_ref[:],
                jnp.zeros((_BM - _TAIL - _B, _D), jnp.float32),
            ],
            axis=0,
        )
        out_ref[:] = _probs_t(x, cent, c2)


def kernel(feats, reservoir_feats, delta_centroids, init_style):
    grid = _LAST + 1
    out_t = pl.pallas_call(
        _body,
        grid=(grid,),
        in_specs=[
            pl.BlockSpec((_BM, _D), lambda i: (i, 0)),
            pl.BlockSpec((_B, _D), lambda i: (0, 0)),
            pl.BlockSpec((_K, _D), lambda i: (0, 0)),
            pl.BlockSpec((1, _D), lambda i: (0, 0)),
        ],
        out_specs=pl.BlockSpec((_K, _BM), lambda i: (0, i)),
        out_shape=jax.ShapeDtypeStruct((_K, _N + _B), jnp.float32),
        compiler_params=pltpu.CompilerParams(
            dimension_semantics=(pltpu.PARALLEL,)
        ),
    )(reservoir_feats, feats, delta_centroids, init_style)
    return out_t.T
